# Initial kernel scaffold; baseline (speedup 1.0000x reference)
import functools

import jax
import jax.numpy as jnp
from jax import lax
from jax.experimental import pallas as pl
from jax.experimental.pallas import tpu as pltpu
from jax.experimental.pallas import tpu_sc as plsc

N = 10000
E = 320000
F = 128
NC = 2
NS = 16
NW = NC * NS
EPW = E // NW
C = 80
NCHUNK = EPW // C
RPT = N // NS
HW = 16

_mesh = plsc.VectorSubcoreMesh(core_axis_name="c", subcore_axis_name="s")


@functools.partial(
    pl.kernel,
    out_type=jax.ShapeDtypeStruct((NC, N, HW), jnp.float32),
    mesh=_mesh,
    scratch_types=[
        pltpu.VMEM((C,), jnp.int32),
        pltpu.VMEM((C, HW), jnp.float32),
        pltpu.VMEM((RPT, HW), jnp.float32),
        pltpu.VMEM_SHARED((N, HW), jnp.float32),
    ],
)
def _sc_degree(row_hbm, out_hbm, idx_v, ones_v, zbuf_v, hist_sh):
    c = lax.axis_index("c")
    s = lax.axis_index("s")
    w = c * NS + s
    ones16 = jnp.full((16,), 1.0, jnp.float32)
    zeros16 = jnp.zeros((16,), jnp.float32)

    @pl.loop(0, C)
    def _(i):
        ones_v[i, :] = ones16

    @pl.loop(0, RPT)
    def _(i):
        zbuf_v[i, :] = zeros16

    pltpu.sync_copy(zbuf_v, hist_sh.at[pl.ds(s * RPT, RPT)])
    plsc.subcore_barrier()

    base = w * EPW

    @pl.loop(0, NCHUNK)
    def _(i):
        pltpu.sync_copy(row_hbm.at[pl.ds(base + i * C, C)], idx_v)
        pltpu.sync_copy(ones_v, hist_sh.at[idx_v], add=True)

    plsc.subcore_barrier()
    pltpu.sync_copy(hist_sh.at[pl.ds(s * RPT, RPT)],
                    out_hbm.at[c, pl.ds(s * RPT, RPT)])


@functools.partial(
    pl.kernel,
    out_type=jax.ShapeDtypeStruct((NC, N, F), jnp.float32),
    mesh=_mesh,
    scratch_types=[
        pltpu.VMEM((C,), jnp.int32),
        pltpu.VMEM((C,), jnp.int32),
        pltpu.VMEM((C, F), jnp.float32),
        pltpu.VMEM((125, F), jnp.float32),
        pltpu.VMEM_SHARED((N, F), jnp.float32),
        pltpu.SemaphoreType.DMA,
    ],
)
def _sc_aggregate(z_hbm, row_hbm, col_hbm, out_hbm,
                  cidx_v, ridx_v, rows_v, zbuf_v, acc_sh, sem):
    c = lax.axis_index("c")
    s = lax.axis_index("s")
    w = c * NS + s
    zeros16 = jnp.zeros((16,), jnp.float32)

    @pl.loop(0, 125)
    def _(i):
        @pl.loop(0, F // 16)
        def _(j):
            zbuf_v[i, pl.ds(j * 16, 16)] = zeros16

    @pl.loop(0, RPT // 125)
    def _(j):
        pltpu.sync_copy(zbuf_v, acc_sh.at[pl.ds(s * RPT + j * 125, 125)])

    plsc.subcore_barrier()

    base = w * EPW

    @pl.loop(0, NCHUNK)
    def _(i):
        pltpu.sync_copy(col_hbm.at[pl.ds(base + i * C, C)], cidx_v)
        pltpu.async_copy(z_hbm.at[cidx_v], rows_v, sem).wait()
        pltpu.sync_copy(row_hbm.at[pl.ds(base + i * C, C)], ridx_v)
        pltpu.sync_copy(rows_v, acc_sh.at[ridx_v], add=True)

    plsc.subcore_barrier()
    pltpu.sync_copy(acc_sh.at[pl.ds(s * RPT, RPT)],
                    out_hbm.at[c, pl.ds(s * RPT, RPT)])


def _mm_body(x_ref, w_ref, o_ref):
    o_ref[...] = jnp.dot(x_ref[...], w_ref[...],
                         preferred_element_type=jnp.float32)


def _tc_matmul(x, weight):
    return pl.pallas_call(
        _mm_body,
        out_shape=jax.ShapeDtypeStruct((N, F), jnp.float32),
    )(x, weight)


def _scale_body(xw_ref, h_ref, z_ref):
    deg = 2.0 + h_ref[0, :, 0:1] + h_ref[1, :, 0:1]
    z_ref[...] = xw_ref[...] * lax.rsqrt(deg)


def _tc_scale(xw, hist):
    return pl.pallas_call(
        _scale_body,
        out_shape=jax.ShapeDtypeStruct((N, F), jnp.float32),
    )(xw, hist)


def _final_body(xw_ref, h_ref, p_ref, o_ref):
    deg = 2.0 + h_ref[0, :, 0:1] + h_ref[1, :, 0:1]
    dinv = lax.rsqrt(deg)
    agg = p_ref[0] + p_ref[1]
    out = dinv * agg + (2.0 / deg) * xw_ref[...]
    o_ref[...] = jnp.maximum(out, 0.0)


def _tc_final(xw, hist, parts):
    return pl.pallas_call(
        _final_body,
        out_shape=jax.ShapeDtypeStruct((N, F), jnp.float32),
    )(xw, hist, parts)


def kernel(x, edge_index, weight):
    row = edge_index[0].astype(jnp.int32)
    col = edge_index[1].astype(jnp.int32)
    xw = _tc_matmul(x, weight)
    hist = _sc_degree(row)
    z = _tc_scale(xw, hist)
    parts = _sc_aggregate(z, row, col)
    return _tc_final(xw, hist, parts)


# speedup vs baseline: 19.9543x; 19.9543x over previous
import functools

import jax
import jax.numpy as jnp
from jax import lax
from jax.experimental import pallas as pl
from jax.experimental.pallas import tpu as pltpu
from jax.experimental.pallas import tpu_sc as plsc

N = 10000
E = 320000
F = 128
NC = 2
NS = 16
NW = NC * NS
EPW = E // NW
C = 80
NCHUNK = EPW // C
NPAD = 10240
RPT = NPAD // NS
HW = 16

_mesh = plsc.VectorSubcoreMesh(core_axis_name="c", subcore_axis_name="s")


GR = NPAD // 16
HSZ = GR * 128


@functools.partial(
    pl.kernel,
    out_type=jax.ShapeDtypeStruct((NW * HSZ,), jnp.float32),
    mesh=_mesh,
    scratch_types=[
        pltpu.VMEM((EPW,), jnp.int32),
        pltpu.VMEM((HSZ,), jnp.float32),
    ],
    compiler_params=pltpu.CompilerParams(needs_layout_passes=False),
)
def _sc_degree(row_hbm, out_hbm, idxs_v, hist_v):
    c = lax.axis_index("c")
    s = lax.axis_index("s")
    w = c * NS + s
    iota = lax.iota(jnp.int32, 16)
    zeros16 = jnp.zeros((16,), jnp.float32)
    ones16 = jnp.full((16,), 1.0, jnp.float32)

    @pl.loop(0, HSZ // 16)
    def _(i):
        hist_v[pl.ds(i * 16, 16)] = zeros16

    pltpu.sync_copy(row_hbm.at[pl.ds(w * EPW, EPW)], idxs_v)
    lane8 = iota & 7
    m_lo = iota < 8
    m_hi = iota >= 8

    @pl.loop(0, EPW // 16)
    def _(i):
        nid = idxs_v[pl.ds(i * 16, 16)]
        flat = nid * 8 + lane8
        plsc.addupdate_scatter(hist_v, [flat], ones16, mask=m_lo)
        plsc.addupdate_scatter(hist_v, [flat], ones16, mask=m_hi)

    pltpu.sync_copy(hist_v, out_hbm.at[pl.ds(w * HSZ, HSZ)])


@functools.partial(
    pl.kernel,
    out_type=jax.ShapeDtypeStruct((NC, NPAD, F), jnp.float32),
    mesh=_mesh,
    scratch_types=[
        pltpu.VMEM((C,), jnp.int32),
        pltpu.VMEM((C,), jnp.int32),
        pltpu.VMEM((C, F), jnp.float32),
        pltpu.VMEM((128, F), jnp.float32),
        pltpu.VMEM_SHARED((NPAD, F), jnp.float32),
        pltpu.SemaphoreType.DMA,
    ],
)
def _sc_aggregate(z_hbm, row_hbm, col_hbm, out_hbm,
                  cidx_v, ridx_v, rows_v, zbuf_v, acc_sh, sem):
    c = lax.axis_index("c")
    s = lax.axis_index("s")
    w = c * NS + s
    zeros16 = jnp.zeros((16,), jnp.float32)

    @pl.loop(0, 128)
    def _(i):
        @pl.loop(0, F // 16)
        def _(j):
            zbuf_v[i, pl.ds(j * 16, 16)] = zeros16

    @pl.loop(0, RPT // 128)
    def _(j):
        pltpu.sync_copy(zbuf_v, acc_sh.at[pl.ds(s * RPT + j * 128, 128)])

    plsc.subcore_barrier()

    base = w * EPW

    @pl.loop(0, NCHUNK)
    def _(i):
        pltpu.sync_copy(col_hbm.at[pl.ds(base + i * C, C)], cidx_v)
        pltpu.async_copy(z_hbm.at[cidx_v], rows_v, sem).wait()
        pltpu.sync_copy(row_hbm.at[pl.ds(base + i * C, C)], ridx_v)
        pltpu.sync_copy(rows_v, acc_sh.at[ridx_v], add=True)

    plsc.subcore_barrier()

    @pl.loop(0, RPT // 128)
    def _(j):
        pltpu.sync_copy(acc_sh.at[pl.ds(s * RPT + j * 128, 128)], zbuf_v)
        pltpu.sync_copy(zbuf_v, out_hbm.at[c, pl.ds(s * RPT + j * 128, 128)])


def _mm_body(x_ref, w_ref, o_ref):
    o_ref[...] = jnp.dot(x_ref[...], w_ref[...],
                         preferred_element_type=jnp.float32)


def _tc_matmul(x, weight):
    return pl.pallas_call(
        _mm_body,
        out_shape=jax.ShapeDtypeStruct((N, F), jnp.float32),
    )(x, weight)


def _deg_from_hist(h_ref):
    sums = jnp.sum(h_ref[...], axis=0)
    j = lax.broadcasted_iota(jnp.int32, (128, 16), 0)
    m = lax.broadcasted_iota(jnp.int32, (128, 16), 1)
    sel = (j // 8 == m).astype(jnp.float32)
    deg = jnp.dot(sums, sel, preferred_element_type=jnp.float32) + 2.0
    return deg[0 : N // 16, :]


def _scale_body(xw_ref, h_ref, z_ref):
    dinv = lax.rsqrt(_deg_from_hist(h_ref))
    xw3 = xw_ref[...].reshape(N // 16, 16, F)
    z_ref[...] = (xw3 * dinv[:, :, None]).reshape(N, F)


def _tc_scale(xw, hist):
    return pl.pallas_call(
        _scale_body,
        out_shape=jax.ShapeDtypeStruct((N, F), jnp.float32),
    )(xw, hist)


def _final_body(xw_ref, h_ref, p_ref, o_ref):
    deg = _deg_from_hist(h_ref)
    dinv = lax.rsqrt(deg)
    agg = (p_ref[0, 0:N, :] + p_ref[1, 0:N, :]).reshape(N // 16, 16, F)
    xw3 = xw_ref[...].reshape(N // 16, 16, F)
    out = dinv[:, :, None] * agg + (2.0 / deg)[:, :, None] * xw3
    o_ref[...] = jnp.maximum(out, 0.0).reshape(N, F)


def _tc_final(xw, hist, parts):
    return pl.pallas_call(
        _final_body,
        out_shape=jax.ShapeDtypeStruct((N, F), jnp.float32),
    )(xw, hist, parts)


def kernel(x, edge_index, weight):
    row = edge_index[0].astype(jnp.int32)
    col = edge_index[1].astype(jnp.int32)
    xw = _tc_matmul(x, weight)
    hist = _sc_degree(row).reshape(NW, GR, 128)
    z = _tc_scale(xw, hist)
    parts = _sc_aggregate(z, row, col)
    return _tc_final(xw, hist, parts)
